# baseline (speedup 1.0000x reference)
import functools

import jax
import jax.numpy as jnp
from jax import lax
from jax.experimental import pallas as pl
from jax.experimental.pallas import tpu as pltpu
from jax.experimental.pallas import tpu_sc as plsc

B = 4096
F = 26
D = 16
DFM = 5
HID = 20
FP = 32
NC, NS = 2, 16
NW = NC * NS
BPW = B // NW
RPW = FP * BPW
NSTR = RPW // 128
CH = 4
RPC = RPW // CH
ORC = RPC // 8


def _sc_gather(table, idx_r):
  mesh = plsc.VectorSubcoreMesh(core_axis_name="c", subcore_axis_name="s",
                                num_cores=NC, num_subcores=NS)

  @functools.partial(
      pl.kernel,
      out_type=jax.ShapeDtypeStruct((B * FP // 8, 8 * D), jnp.float32),
      mesh=mesh,
      compiler_params=pltpu.CompilerParams(use_tc_tiling_on_sc=False),
      scratch_types=[
          pltpu.VMEM((NSTR, 128), jnp.int32),
          pltpu.VMEM((2, RPC, D), jnp.float32),
          pltpu.VMEM((2, ORC, 8 * D), jnp.float32),
          pltpu.SemaphoreType.DMA,
      ],
  )
  def gather(idx_hbm, table_hbm, out_hbm, idx_v, buf, buf2, sem):
    wid = lax.axis_index("s") * NC + lax.axis_index("c")
    pltpu.sync_copy(idx_hbm.at[wid], idx_v)

    def fire(c):
      ph = lax.rem(c, 2)
      for k in range(8):
        pltpu.async_copy(table_hbm.at[idx_v.at[c * 8 + k]],
                         buf.at[ph, pl.ds(k * 128, 128)], sem)

    fire(0)
    def chunk(c, carry):
      ph = lax.rem(c, 2)
      @pl.when(c + 1 < CH)
      def _():
        fire(c + 1)
      for k in range(8):
        pltpu.make_async_copy(table_hbm.at[idx_v.at[c * 8 + k]],
                              buf.at[ph, pl.ds(k * 128, 128)], sem).wait()
      def rl(rr, carry2):
        for k in range(8):
          buf2[ph, rr, pl.ds(k * D, D)] = buf[ph, rr * 8 + k, :]
        return carry2
      lax.fori_loop(0, ORC, rl, 0)
      pltpu.sync_copy(buf2.at[ph],
                      out_hbm.at[pl.ds(wid * (RPW // 8) + c * ORC, ORC), :])
      return carry
    lax.fori_loop(0, CH, chunk, 0)

  return gather(idx_r, table)


def _tc_dense(emb512, dense_features, labels2, G512, M512, Wd0, W_dense,
              b_dense, b_deep, W1a, w1row, b1, W2, b2):
  def body(emb_ref, dense_ref, lab_ref, g_ref, m_ref, wd0_ref, wdn_ref,
           bdn_ref, bdp_ref, w1_ref, w1r_ref, b1_ref, w2_ref, b2_ref,
           loss_ref, p_ref):
    e = emb_ref[...]
    acc = jnp.dot(e, g_ref[...], preferred_element_type=jnp.float32)
    sq = jnp.dot(e * e, m_ref[...], preferred_element_type=jnp.float32)
    dense_e = jnp.maximum(
        jnp.dot(dense_ref[...], wdn_ref[...],
                preferred_element_type=jnp.float32) + bdn_ref[...], 0.0)
    s = acc[:, 0:D] + dense_e
    deep = jnp.maximum(
        acc[:, D:D + DFM]
        + jnp.dot(dense_e, wd0_ref[...], preferred_element_type=jnp.float32)
        + bdp_ref[...], 0.0)
    fmv = s * s - (sq + dense_e * dense_e)
    fm = 0.5 * jnp.dot(fmv, jnp.ones((D, 1), jnp.float32),
                       preferred_element_type=jnp.float32)
    h = jnp.maximum(
        jnp.dot(deep, w1_ref[...], preferred_element_type=jnp.float32)
        + fm * w1r_ref[...] + b1_ref[...], 0.0)
    logits = jnp.dot(h, w2_ref[...],
                     preferred_element_type=jnp.float32) + b2_ref[...]
    p = 1.0 / (1.0 + jnp.exp(-logits))
    p = jnp.clip(p, 1e-7, 1.0 - 1e-7)
    lab = lab_ref[...]
    ll = lab * jnp.log(p) + (1.0 - lab) * jnp.log(1.0 - p)
    loss_ref[...] = jnp.broadcast_to(-jnp.sum(ll) * (1.0 / B), (1, 1))
    p_ref[...] = p

  return pl.pallas_call(
      body,
      out_shape=(jax.ShapeDtypeStruct((1, 1), jnp.float32),
                 jax.ShapeDtypeStruct((B, 1), jnp.float32)),
  )(emb512, dense_features, labels2, G512, M512, Wd0, W_dense,
    b_dense, b_deep, W1a, w1row, b1, W2, b2)


def kernel(dense_features, sparse_features, permu, labels, table, W_dense,
           b_dense, W_deep, b_deep, W_over1, b_over1, W_over2, b_over2):
  idx = jnp.take(sparse_features, permu, axis=1).astype(jnp.int32)
  idx = jnp.pad(idx, ((0, 0), (0, FP - F)))
  idx_r = idx.reshape(NW, NSTR, 128)

  rows = _sc_gather(table, idx_r)
  emb512 = rows.reshape(B, FP * D)

  Wd_emb = W_deep[D:(F + 1) * D].reshape(F, D, DFM)
  Wd512 = jnp.pad(Wd_emb, ((0, FP - F), (0, 0), (0, 0))).reshape(FP * D, DFM)
  eye = jnp.broadcast_to(jnp.eye(D, dtype=jnp.float32)[None], (FP, D, D))
  msk = (jnp.arange(FP) < F).astype(jnp.float32)[:, None, None]
  M512 = (eye * msk).reshape(FP * D, D)
  G512 = jnp.concatenate([M512, Wd512], axis=1)

  loss, p = _tc_dense(
      emb512, dense_features, labels.reshape(B, 1), G512, M512,
      W_deep[0:D], W_dense, b_dense.reshape(1, D), b_deep.reshape(1, DFM),
      W_over1[0:DFM], W_over1[DFM:DFM + 1], b_over1.reshape(1, HID),
      W_over2, b_over2.reshape(1, 1))
  return (loss.reshape(()), p.reshape(B), labels)


# speedup vs baseline: 1.2374x; 1.2374x over previous
import functools

import jax
import jax.numpy as jnp
from jax import lax
from jax.experimental import pallas as pl
from jax.experimental.pallas import tpu as pltpu
from jax.experimental.pallas import tpu_sc as plsc

B = 4096
F = 26
D = 16
DFM = 5
HID = 20
FP = 32
NC, NS = 2, 16
NW = NC * NS
BPW = B // NW
RPW = F * BPW
NSTR = RPW // 128
CH = 4
BPC = BPW // CH
NG = FP * D // 128


def _sc_gather(table, idx_r):
  mesh = plsc.VectorSubcoreMesh(core_axis_name="c", subcore_axis_name="s",
                                num_cores=NC, num_subcores=NS)

  @functools.partial(
      pl.kernel,
      out_type=jax.ShapeDtypeStruct((NG, B, 8 * D), jnp.float32),
      mesh=mesh,
      compiler_params=pltpu.CompilerParams(use_tc_tiling_on_sc=False),
      scratch_types=[
          pltpu.VMEM((NSTR, 128), jnp.int32),
          pltpu.VMEM((RPW, D), jnp.float32),
          pltpu.VMEM((2, NG, BPC, 8 * D), jnp.float32),
          [pltpu.SemaphoreType.DMA] * 8,
      ],
  )
  def gather(idx_hbm, table_hbm, out_hbm, idx_v, buf, buf2, sems):
    wid = lax.axis_index("s") * NC + lax.axis_index("c")
    pltpu.sync_copy(idx_hbm.at[wid], idx_v)

    for t in range(NSTR):
      pltpu.async_copy(table_hbm.at[idx_v.at[t]],
                       buf.at[pl.ds(t * 128, 128)], sems[t % 8])
    for t in range(NSTR):
      pltpu.make_async_copy(table_hbm.at[idx_v.at[t]],
                            buf.at[pl.ds(t * 128, 128)], sems[t % 8]).wait()

    zero = jnp.zeros((D,), jnp.float32)
    def chunk(c, carry):
      ph = lax.rem(c, 2)
      def rl(rr, carry2):
        for q in range(NG):
          for k in range(8):
            f = 8 * q + k
            if f < F:
              buf2[ph, q, rr, pl.ds(k * D, D)] = \
                  buf[(c * BPC + rr) * F + f, :]
            else:
              buf2[ph, q, rr, pl.ds(k * D, D)] = zero
        return carry2
      lax.fori_loop(0, BPC, rl, 0)
      for q in range(NG):
        pltpu.sync_copy(
            buf2.at[ph, q],
            out_hbm.at[q, pl.ds(wid * BPW + c * BPC, BPC), :])
      return carry
    lax.fori_loop(0, CH, chunk, 0)

  return gather(idx_r, table)


def _tc_dense(emb_q, dense_features, labels2, G4, Wd0, W_dense,
              b_dense, b_deep, W1a, w1row, b1, W2, b2):
  def body(emb_ref, dense_ref, lab_ref, g_ref, wd0_ref, wdn_ref,
           bdn_ref, bdp_ref, w1_ref, w1r_ref, b1_ref, w2_ref, b2_ref,
           loss_ref, p_ref):
    acc = None
    sq = None
    for q in range(NG):
      e = emb_ref[q]
      a_q = jnp.dot(e, g_ref[q], preferred_element_type=jnp.float32)
      s_q = jnp.dot(e * e, g_ref[q, :, 0:D],
                    preferred_element_type=jnp.float32)
      acc = a_q if acc is None else acc + a_q
      sq = s_q if sq is None else sq + s_q
    dense_e = jnp.maximum(
        jnp.dot(dense_ref[...], wdn_ref[...],
                preferred_element_type=jnp.float32) + bdn_ref[...], 0.0)
    s = acc[:, 0:D] + dense_e
    deep = jnp.maximum(
        acc[:, D:D + DFM]
        + jnp.dot(dense_e, wd0_ref[...], preferred_element_type=jnp.float32)
        + bdp_ref[...], 0.0)
    fmv = s * s - (sq + dense_e * dense_e)
    fm = 0.5 * jnp.dot(fmv, jnp.ones((D, 1), jnp.float32),
                       preferred_element_type=jnp.float32)
    h = jnp.maximum(
        jnp.dot(deep, w1_ref[...], preferred_element_type=jnp.float32)
        + fm * w1r_ref[...] + b1_ref[...], 0.0)
    logits = jnp.dot(h, w2_ref[...],
                     preferred_element_type=jnp.float32) + b2_ref[...]
    p = 1.0 / (1.0 + jnp.exp(-logits))
    p = jnp.clip(p, 1e-7, 1.0 - 1e-7)
    lab = lab_ref[...]
    ll = lab * jnp.log(p) + (1.0 - lab) * jnp.log(1.0 - p)
    loss_ref[...] = jnp.broadcast_to(-jnp.sum(ll) * (1.0 / B), (1, 1))
    p_ref[...] = p

  return pl.pallas_call(
      body,
      out_shape=(jax.ShapeDtypeStruct((1, 1), jnp.float32),
                 jax.ShapeDtypeStruct((B, 1), jnp.float32)),
  )(emb_q, dense_features, labels2, G4, Wd0, W_dense,
    b_dense, b_deep, W1a, w1row, b1, W2, b2)


def kernel(dense_features, sparse_features, permu, labels, table, W_dense,
           b_dense, W_deep, b_deep, W_over1, b_over1, W_over2, b_over2):
  idx = jnp.take(sparse_features, permu, axis=1).astype(jnp.int32)
  idx_r = idx.reshape(NW, NSTR, 128)

  emb_q = _sc_gather(table, idx_r)

  Wd_emb = W_deep[D:(F + 1) * D].reshape(F, D, DFM)
  Wd512 = jnp.pad(Wd_emb, ((0, FP - F), (0, 0), (0, 0))).reshape(FP * D, DFM)
  eye = jnp.broadcast_to(jnp.eye(D, dtype=jnp.float32)[None], (FP, D, D))
  msk = (jnp.arange(FP) < F).astype(jnp.float32)[:, None, None]
  M512 = (eye * msk).reshape(FP * D, D)
  G4 = jnp.concatenate([M512, Wd512], axis=1).reshape(NG, 128, D + DFM)

  loss, p = _tc_dense(
      emb_q, dense_features, labels.reshape(B, 1), G4,
      W_deep[0:D], W_dense, b_dense.reshape(1, D), b_deep.reshape(1, DFM),
      W_over1[0:DFM], W_over1[DFM:DFM + 1], b_over1.reshape(1, HID),
      W_over2, b_over2.reshape(1, 1))
  return (loss.reshape(()), p.reshape(B), labels)
